# Initial kernel scaffold; baseline (speedup 1.0000x reference)
import jax
import jax.numpy as jnp
from jax.experimental import pallas as pl


def kernel(x, Wsw, bsw, W1, b1, W2, b2):
    raise NotImplementedError("write your pallas kernel here")


# speedup vs baseline: 1.2288x; 1.2288x over previous
import functools

import jax
import jax.numpy as jnp
from jax import lax
from jax.experimental import pallas as pl
from jax.experimental.pallas import tpu as pltpu
from jax.experimental.pallas import tpu_sc as plsc

_E = 8
_DM = 1024
_DF = 4096
_S = 2048
_TS = 256
_TMAX = 16
_P = _TMAX * _TS
_NW = 32


def _router_body(x_ref, w_ref, b_ref, r_ref, g_ref):
    l = jnp.dot(x_ref[...], w_ref[...]) + b_ref[...]
    m = jnp.max(l, axis=1, keepdims=True)
    idx = lax.broadcasted_iota(jnp.int32, l.shape, 1)
    r_ref[...] = jnp.min(jnp.where(l == m, idx, _E), axis=1, keepdims=True)
    g_ref[...] = 1.0 / jnp.sum(jnp.exp(l - m), axis=1, keepdims=True)


def _router(xf, Wsw, bsw):
    r2, g2 = pl.pallas_call(
        _router_body,
        out_shape=(
            jax.ShapeDtypeStruct((_S, 1), jnp.int32),
            jax.ShapeDtypeStruct((_S, 1), jnp.float32),
        ),
    )(xf, Wsw, bsw.reshape(1, _E))
    return r2[:, 0], g2[:, 0]


_NF = 2
_FC = _DF // _NF


def _ffn_body(eot_ref, na_ref, xs_ref, w1_ref, b1_ref, w2_ref, b2_ref, g_ref,
              o_ref):
    t = pl.program_id(0)
    f = pl.program_id(1)

    @pl.when(t < na_ref[0])
    def _():
        h = jnp.dot(xs_ref[...], w1_ref[0]) + b1_ref[0]
        h = jnp.maximum(h, 0.0)
        part = jnp.dot(h, w2_ref[0]) * g_ref[...]

        @pl.when(f == 0)
        def _():
            o_ref[...] = part + b2_ref[0] * g_ref[...]

        @pl.when(f != 0)
        def _():
            o_ref[...] += part


def _ffn(eot, nact, xs, W1, b1, W2, b2, gs):
    grid_spec = pltpu.PrefetchScalarGridSpec(
        num_scalar_prefetch=2,
        grid=(_TMAX, _NF),
        in_specs=[
            pl.BlockSpec((_TS, _DM), lambda t, f, eot, na: (t, 0)),
            pl.BlockSpec((1, _DM, _FC), lambda t, f, eot, na: (eot[t], 0, f)),
            pl.BlockSpec((1, 1, _FC), lambda t, f, eot, na: (eot[t], 0, f)),
            pl.BlockSpec((1, _FC, _DM), lambda t, f, eot, na: (eot[t], f, 0)),
            pl.BlockSpec((1, 1, _DM), lambda t, f, eot, na: (eot[t], 0, 0)),
            pl.BlockSpec((_TS, 1), lambda t, f, eot, na: (t, 0)),
        ],
        out_specs=pl.BlockSpec((_TS, _DM), lambda t, f, eot, na: (t, 0)),
    )
    return pl.pallas_call(
        _ffn_body,
        grid_spec=grid_spec,
        out_shape=jax.ShapeDtypeStruct((_P, _DM), jnp.float32),
        compiler_params=pltpu.CompilerParams(
            dimension_semantics=("arbitrary", "arbitrary")),
    )(eot, nact, xs, W1, b1.reshape(_E, 1, _DF), W2,
      b2.reshape(_E, 1, _DM), gs)


def _sc_gather(table, idx, ch):
    _, d = table.shape
    b = idx.shape[0]
    bpw = b // _NW
    nch = bpw // ch
    mesh = plsc.VectorSubcoreMesh(core_axis_name="c", subcore_axis_name="s")

    @functools.partial(
        pl.kernel,
        out_type=jax.ShapeDtypeStruct((b, d), jnp.float32),
        mesh=mesh,
        scratch_types=[
            pltpu.VMEM((ch,), jnp.int32),
            pltpu.VMEM((ch, d), jnp.float32),
            pltpu.SemaphoreType.DMA,
        ],
    )
    def k(table_hbm, idx_hbm, out_hbm, idx_v, rows_v, sem):
        wid = lax.axis_index("s") * 2 + lax.axis_index("c")
        base = wid * bpw
        for c in range(nch):
            off = base + c * ch
            pltpu.sync_copy(idx_hbm.at[pl.ds(off, ch)], idx_v)
            pltpu.async_copy(table_hbm.at[idx_v], rows_v, sem).wait()
            pltpu.sync_copy(rows_v, out_hbm.at[pl.ds(off, ch)])

    return k(table, idx)


def kernel(x, Wsw, bsw, W1, b1, W2, b2):
    bsz, seq, dm = x.shape
    xf = x.reshape(_S, _DM)

    routes, gates = _router(xf, Wsw, bsw)

    i32 = jnp.int32
    onehot = (routes[None, :] == jnp.arange(_E, dtype=i32)[:, None])
    csum = jnp.cumsum(onehot.astype(i32), axis=1)
    counts = csum[:, -1]
    rank = jnp.take_along_axis(csum, routes[None, :], axis=0)[0] - 1
    tile_cnt = (counts + _TS - 1) // _TS
    cum_tiles = jnp.cumsum(tile_cnt)
    nact = cum_tiles[-1].astype(i32)
    group_start = (cum_tiles - tile_cnt) * _TS
    dest = group_start[routes] + rank
    tok_ids = jnp.zeros((_P,), i32).at[dest].set(jnp.arange(_S, dtype=i32))
    gs = jnp.zeros((_P,), jnp.float32).at[dest].set(gates).reshape(_P, 1)
    t_idx = jnp.arange(_TMAX, dtype=i32)
    eot_raw = jnp.sum(t_idx[:, None] >= cum_tiles[None, :], axis=1)
    e_last = eot_raw[nact - 1]
    eot = jnp.where(t_idx < nact, eot_raw, e_last).astype(i32)

    xs = _sc_gather(xf, tok_ids, 64)
    out_sorted = _ffn(eot, nact[None], xs, W1, b1, W2, b2, gs)
    final = _sc_gather(out_sorted, dest.astype(i32), 64)
    return final.reshape(bsz, seq, dm)


# speedup vs baseline: 1.6865x; 1.3725x over previous
import functools

import jax
import jax.numpy as jnp
from jax import lax
from jax.experimental import pallas as pl
from jax.experimental.pallas import tpu as pltpu
from jax.experimental.pallas import tpu_sc as plsc

_E = 8
_DM = 1024
_DF = 4096
_S = 2048
_TS = 256
_TMAX = 16
_P = _TMAX * _TS
_NW = 32


def _router_body(x_ref, w_ref, b_ref, r_ref, g_ref):
    l = jnp.dot(x_ref[...], w_ref[...]) + b_ref[...]
    m = jnp.max(l, axis=1, keepdims=True)
    idx = lax.broadcasted_iota(jnp.int32, l.shape, 1)
    r_ref[...] = jnp.min(jnp.where(l == m, idx, _E), axis=1, keepdims=True)
    g_ref[...] = 1.0 / jnp.sum(jnp.exp(l - m), axis=1, keepdims=True)


def _router(xf, Wsw, bsw):
    r2, g2 = pl.pallas_call(
        _router_body,
        out_shape=(
            jax.ShapeDtypeStruct((_S, 1), jnp.int32),
            jax.ShapeDtypeStruct((_S, 1), jnp.float32),
        ),
    )(xf, Wsw, bsw.reshape(1, _E))
    return r2[:, 0], g2[:, 0]


_NF = 2
_FC = _DF // _NF


def _ffn_body(eot_ref, na_ref, xs_ref, w1_ref, b1_ref, w2_ref, b2_ref, g_ref,
              o_ref):
    t = pl.program_id(0)
    f = pl.program_id(1)

    @pl.when(t < na_ref[0])
    def _():
        h = jnp.dot(xs_ref[...], w1_ref[0]) + b1_ref[0]
        h = jnp.maximum(h, 0.0)
        part = jnp.dot(h, w2_ref[0]) * g_ref[...]

        @pl.when(f == 0)
        def _():
            o_ref[...] = part + b2_ref[0] * g_ref[...]

        @pl.when(f != 0)
        def _():
            o_ref[...] += part


def _ffn(eot, nact, xs, W1, b1, W2, b2, gs):
    grid_spec = pltpu.PrefetchScalarGridSpec(
        num_scalar_prefetch=2,
        grid=(_TMAX, _NF),
        in_specs=[
            pl.BlockSpec((_TS, _DM), lambda t, f, eot, na: (t, 0)),
            pl.BlockSpec((1, _DM, _FC), lambda t, f, eot, na: (eot[t], 0, f)),
            pl.BlockSpec((1, 1, _FC), lambda t, f, eot, na: (eot[t], 0, f)),
            pl.BlockSpec((1, _FC, _DM), lambda t, f, eot, na: (eot[t], f, 0)),
            pl.BlockSpec((1, 1, _DM), lambda t, f, eot, na: (eot[t], 0, 0)),
            pl.BlockSpec((_TS, 1), lambda t, f, eot, na: (t, 0)),
        ],
        out_specs=pl.BlockSpec((_TS, _DM), lambda t, f, eot, na: (t, 0)),
    )
    return pl.pallas_call(
        _ffn_body,
        grid_spec=grid_spec,
        out_shape=jax.ShapeDtypeStruct((_P, _DM), jnp.float32),
        compiler_params=pltpu.CompilerParams(
            dimension_semantics=("arbitrary", "arbitrary")),
    )(eot, nact, xs, W1, b1.reshape(_E, 1, _DF), W2,
      b2.reshape(_E, 1, _DM), gs)


def _sc_gather(table, idx, ch):
    _, d = table.shape
    b = idx.shape[0]
    bpw = b // _NW
    nch = bpw // ch
    mesh = plsc.VectorSubcoreMesh(core_axis_name="c", subcore_axis_name="s")

    @functools.partial(
        pl.kernel,
        out_type=jax.ShapeDtypeStruct((b, d), jnp.float32),
        mesh=mesh,
        scratch_types=[
            pltpu.VMEM((ch,), jnp.int32),
            pltpu.VMEM((ch, d), jnp.float32),
            pltpu.SemaphoreType.DMA,
        ],
    )
    def k(table_hbm, idx_hbm, out_hbm, idx_v, rows_v, sem):
        wid = lax.axis_index("s") * 2 + lax.axis_index("c")
        base = wid * bpw
        for c in range(nch):
            off = base + c * ch
            pltpu.sync_copy(idx_hbm.at[pl.ds(off, ch)], idx_v)
            pltpu.async_copy(table_hbm.at[idx_v], rows_v, sem).wait()
            pltpu.sync_copy(rows_v, out_hbm.at[pl.ds(off, ch)])

    return k(table, idx)


def kernel(x, Wsw, bsw, W1, b1, W2, b2):
    bsz, seq, dm = x.shape
    xf = x.reshape(_S, _DM)

    routes, gates = _router(xf, Wsw, bsw)

    i32 = jnp.int32
    onehot = (routes[None, :] == jnp.arange(_E, dtype=i32)[:, None])
    csum = jnp.cumsum(onehot.astype(i32), axis=1)
    counts = csum[:, -1]
    rank = jnp.take_along_axis(csum, routes[None, :], axis=0)[0] - 1
    tile_cnt = (counts + _TS - 1) // _TS
    cum_tiles = jnp.cumsum(tile_cnt)
    nact = cum_tiles[-1].astype(i32)
    group_start = (cum_tiles - tile_cnt) * _TS
    dest = group_start[routes] + rank
    fill = jnp.arange(_P, dtype=i32) % _S
    tok_ids = fill.at[dest].set(jnp.arange(_S, dtype=i32))
    gs = jnp.zeros((_P,), jnp.float32).at[dest].set(gates).reshape(_P, 1)
    t_idx = jnp.arange(_TMAX, dtype=i32)
    eot_raw = jnp.sum(t_idx[:, None] >= cum_tiles[None, :], axis=1)
    e_last = eot_raw[nact - 1]
    eot = jnp.where(t_idx < nact, eot_raw, e_last).astype(i32)

    xs = _sc_gather(xf, tok_ids, 64)
    out_sorted = _ffn(eot, nact[None], xs, W1, b1, W2, b2, gs)
    final = _sc_gather(out_sorted, dest.astype(i32), 64)
    return final.reshape(bsz, seq, dm)
